# baseline (speedup 1.0000x reference)
import jax
import jax.numpy as jnp
from jax import lax
from jax.experimental import pallas as pl
from jax.experimental.pallas import tpu as pltpu
from jax.experimental.pallas import tpu_sc as plsc

NUM_EMB = 1000000
DIM = 32
BATCH = 4096
HIST = 200

NC = 2
NS = 16
NW = NC * NS

BPW = BATCH // NW
NBUF = 2
ROUNDS = HIST // NBUF


def _emb_body(tok_hbm, w_hbm, out_hbm, idxg_v, sub_v, g0, g1, o0, o1,
              gs0, gs1, os0, os1):
    gbufs = (g0, g1)
    obufs = (o0, o1)
    gsems = (gs0, gs1)
    osems = (os0, os1)

    cid = lax.axis_index("c")
    sid = lax.axis_index("s")
    w = sid * NC + cid

    pltpu.sync_copy(tok_hbm.at[w], idxg_v)

    def prep(h, carry):
        for k in range(BPW // 16):
            sl = pl.ds(k * 16, 16)
            v = idxg_v[h, sl]
            sub_v[h, sl] = (v & 3) * 32
            idxg_v[h, sl] = v >> 2
        return carry

    lax.fori_loop(0, HIST, prep, 0)

    def fire_gather(h, b):
        pltpu.async_copy(w_hbm.at[idxg_v.at[h]], gbufs[b], gsems[b])

    def drain_gather(h, b):
        pltpu.make_async_copy(w_hbm.at[idxg_v.at[h]], gbufs[b],
                              gsems[b]).wait()

    def fire_writes(h, b):
        for g in range(4):
            base = h * 1024 + g * 256 + w * 8
            pltpu.async_copy(obufs[b].at[pl.ds(8 * g, 8)],
                             out_hbm.at[pl.ds(base, 8)], osems[b])

    def drain_writes(b):
        pltpu.make_async_copy(obufs[b], out_hbm.at[pl.ds(0, DIM)],
                              osems[b]).wait()

    def extract(h, b):
        for k in range(BPW // 16):
            sl = pl.ds(k * 16, 16)
            rowv = lax.iota(jnp.int32, 16) + (k * 16)
            colb = sub_v[h, sl]
            for c in range(DIM):
                vals = plsc.load_gather(gbufs[b], [rowv, colb + c])
                obufs[b][c, sl] = vals

    for b in range(NBUF):
        fire_gather(b, b)

    def round_body(r, carry):
        h0 = r * NBUF
        for b in range(NBUF):
            h = h0 + b
            drain_gather(h, b)

            @pl.when(r > 0)
            def _():
                drain_writes(b)

            extract(h, b)
            fire_writes(h, b)

            @pl.when(h + NBUF < HIST)
            def _():
                fire_gather(h + NBUF, b)
        return carry

    lax.fori_loop(0, ROUNDS, round_body, 0)
    for b in range(NBUF):
        drain_writes(b)


def kernel(tokens_ids, weights):
    tokA = tokens_ids.T.reshape(HIST, NW, BPW).transpose(1, 0, 2)
    w128 = weights.reshape(NUM_EMB // 4, 4 * DIM)
    out = pl.kernel(
        _emb_body,
        out_type=jax.ShapeDtypeStruct((BATCH * HIST * DIM // 128, 128),
                                      jnp.float32),
        mesh=plsc.VectorSubcoreMesh(
            core_axis_name="c", subcore_axis_name="s",
            num_cores=NC, num_subcores=NS,
        ),
        scratch_types=(
            [pltpu.VMEM((HIST, BPW), jnp.int32),
             pltpu.VMEM((HIST, BPW), jnp.int32)]
            + [pltpu.VMEM((BPW, 4 * DIM), jnp.float32) for _ in range(NBUF)]
            + [pltpu.VMEM((DIM, BPW), jnp.float32) for _ in range(NBUF)]
            + [pltpu.SemaphoreType.DMA for _ in range(2 * NBUF)]
        ),
        compiler_params=pltpu.CompilerParams(use_tc_tiling_on_sc=False,
                                             needs_layout_passes=False),
    )(tokA, w128)
    r = out.reshape(HIST, 4, NW, 8, BPW).transpose(2, 4, 0, 1, 3)
    return r.reshape(BATCH, HIST, DIM)


# speedup vs baseline: 1.2445x; 1.2445x over previous
import jax
import jax.numpy as jnp
from jax import lax
from jax.experimental import pallas as pl
from jax.experimental.pallas import tpu as pltpu
from jax.experimental.pallas import tpu_sc as plsc

NUM_EMB = 1000000
DIM = 32
BATCH = 4096
HIST = 200

NC = 2
NS = 16
NW = NC * NS

BPW = BATCH // NW
NBUF = 2
ROUNDS = HIST // NBUF
OUT_WORDS = BATCH * HIST * DIM


def _emb_body(tok_hbm, w_hbm, out_hbm, idx_v, g0, g1, o0, o1,
              gs0, gs1, os0, os1):
    gbufs = (g0, g1)
    obufs = (o0, o1)
    gsems = (gs0, gs1)
    osems = (os0, os1)

    cid = lax.axis_index("c")
    sid = lax.axis_index("s")
    w = sid * NC + cid

    pltpu.sync_copy(tok_hbm.at[w], idx_v)

    iota128 = lax.iota(jnp.int32, 16) * BPW
    idx8 = [iota128 + d for d in range(8)]

    def fire_gather(h, b):
        pltpu.async_copy(w_hbm.at[idx_v.at[h]], gbufs[b], gsems[b])

    def drain_gather(h, b):
        pltpu.make_async_copy(w_hbm.at[idx_v.at[h]], gbufs[b],
                              gsems[b]).wait()

    def fire_writes(h, b):
        for g in range(4):
            base = (h * 1024 + g * 256 + w * 8) * BPW
            pltpu.async_copy(obufs[b].at[pl.ds(g * 8 * BPW, 8 * BPW)],
                             out_hbm.at[pl.ds(base, 8 * BPW)], osems[b])

    def drain_writes(b):
        pltpu.make_async_copy(obufs[b], out_hbm.at[pl.ds(0, DIM * BPW)],
                              osems[b]).wait()

    def extract(b):
        grp = 8
        for bm0 in range(0, BPW, grp):
            pend = []
            for bm in range(bm0, bm0 + grp):
                for half in range(DIM // 16):
                    v = gbufs[b][bm, pl.ds(half * 16, 16)]
                    pend.append((half, bm, v))
            for half, bm, v in pend:
                off = half * 16 * BPW + (bm & ~7)
                view = obufs[b].at[pl.ds(off, 15 * BPW + 8)]
                plsc.store_scatter(view, [idx8[bm & 7]], v)

    for b in range(NBUF):
        fire_gather(b, b)

    def round_body(r, carry):
        h0 = r * NBUF
        for b in range(NBUF):
            h = h0 + b
            drain_gather(h, b)

            @pl.when(r > 0)
            def _():
                drain_writes(b)

            extract(b)
            fire_writes(h, b)

            @pl.when(h + NBUF < HIST)
            def _():
                fire_gather(h + NBUF, b)
        return carry

    lax.fori_loop(0, ROUNDS, round_body, 0)
    for b in range(NBUF):
        drain_writes(b)


def kernel(tokens_ids, weights):
    tokA = tokens_ids.T.reshape(HIST, NW, BPW).transpose(1, 0, 2)
    out = pl.kernel(
        _emb_body,
        out_type=jax.ShapeDtypeStruct((OUT_WORDS,), jnp.float32),
        mesh=plsc.VectorSubcoreMesh(
            core_axis_name="c", subcore_axis_name="s",
            num_cores=NC, num_subcores=NS,
        ),
        scratch_types=(
            [pltpu.VMEM((HIST, BPW), jnp.int32)]
            + [pltpu.VMEM((BPW, DIM), jnp.float32) for _ in range(NBUF)]
            + [pltpu.VMEM((DIM * BPW,), jnp.float32) for _ in range(NBUF)]
            + [pltpu.SemaphoreType.DMA for _ in range(2 * NBUF)]
        ),
        compiler_params=pltpu.CompilerParams(use_tc_tiling_on_sc=False,
                                             needs_layout_passes=False),
    )(tokA, weights)
    r = out.reshape(HIST, 4, NW, 8, BPW).transpose(2, 4, 0, 1, 3)
    return r.reshape(BATCH, HIST, DIM)
